# baseline (speedup 1.0000x reference)
import functools
import numpy as np
import jax
import jax.numpy as jnp
from jax import lax
from jax.experimental import pallas as pl
from jax.experimental.pallas import tpu as pltpu
from jax.experimental.pallas import tpu_sc as plsc

_FS = 0.0072973525693
_HARTREE = 27.211386245988
_BOHR = 0.529177210903
_XON = 8.0
_XOFF = 10.0

_C1 = float(_FS ** (-4.0 / 21.0))
_B0 = -0.00433008
_B1 = 0.24428889
_B2 = 0.04125273
_B3 = -0.00078893
_KI7 = float((8.0 / 7.0) * (127.0 - 0.0450466) * (2 ** 23))

_NC, _NS, _L = 2, 16, 16
_NW = _NC * _NS
_CHUNK = 1024
_UNROLL = 4

_ALPHAS_TAB = np.linspace(4.5, 400.0, 100, dtype=np.float64).astype(np.float32)
_C6_TAB = np.linspace(6.5, 4000.0, 100, dtype=np.float64).astype(np.float32)


def _inv_root7(x):
    f32 = jnp.float32
    b = lax.bitcast_convert_type(x, jnp.int32).astype(f32)
    z = lax.bitcast_convert_type(
        (f32(_KI7) - b * f32(1.0 / 7.0)).astype(jnp.int32), f32)
    for _ in range(3):
        z2 = z * z
        z4 = z2 * z2
        xz7 = (x * z) * z2 * z4
        z = z * f32(8.0 / 7.0) - (z * xz7) * f32(1.0 / 7.0)
    return z


def _edge_energy(ai, aj, ci, cj, d):
    f32 = jnp.float32
    x = (ai + aj) * f32(0.5)
    c6 = (f32(2.0) * ci * cj * ai * aj) / (ai * ai * cj + aj * aj * ci)
    z = _inv_root7(x)
    z2 = z * z
    z6 = z2 * z2 * z2
    t = x * z6
    vdw = f32(_C1) * t
    sig = ((f32(_B3) * vdw + f32(_B2)) * vdw + f32(_B1)) * vdw + f32(_B0)
    sig2 = sig * sig
    m8 = f32(10.0) * sig2
    m10 = f32(122.5) * sig2 * sig2
    p = f32(5.08) * t
    p2 = p * p
    p4 = p2 * p2
    r = d * f32(1.0 / _BOHR)
    r2 = r * r
    r4 = r2 * r2
    da = r4 * r2 + p4 * p2
    db = r4 * r4 + p4 * p4
    dc = r4 * r4 * r2 + p4 * p4 * p2
    dbdc = db * dc
    poly = dbdc + m8 * (da * dc) + m10 * (da * db)
    den3 = da * dbdc
    cc = (d - f32(_XON)) * f32(1.0 / (_XOFF - _XON))
    x1 = f32(1.0) - cc
    p1 = x1 > 0
    p2m = cc > 0
    x1p = jnp.where(p1, x1, f32(1.0))
    ccp = jnp.where(p2m, cc, f32(1.0))
    q = f32(1.0) / (x1p * ccp)
    s1 = jnp.where(p1, jnp.exp(-ccp * q), f32(0.0))
    s2 = jnp.where(p2m, jnp.exp(-x1p * q), f32(0.0))
    num = (c6 * s1) * poly
    den = den3 * (s1 + s2)
    e = num / den * f32(-0.5 * _HARTREE)
    return jnp.where(d > 0, e, f32(0.0))


def _sc_body(nsl, npad, ew, nch,
             an_hbm, h_hbm, ii_hbm, jj_hbm, dd_hbm, ta_hbm, tc_hbm,
             part_hbm, a_hbm, c_hbm,
             alpha_t, c6_t, ii0_v, ii1_v, ii2_v, jj0_v, jj1_v, jj2_v,
             dd0_v, dd1_v, dd2_v, ee0_v, ee1_v, ee2_v, pbuf, sems, ssems,
             accum):
    ii_b = (ii0_v, ii1_v, ii2_v)
    jj_b = (jj0_v, jj1_v, jj2_v)
    dd_b = (dd0_v, dd1_v, dd2_v)
    ee_b = (ee0_v, ee1_v, ee2_v)
    c = lax.axis_index("c")
    s = lax.axis_index("s")
    f32 = jnp.float32
    nbase = s * nsl
    cbase = c * npad

    nss = nsl // 2

    def phase1(tab_a_v, tab_c_v, an_v, h_v, sa_v, sc_v):
        pltpu.sync_copy(ta_hbm, tab_a_v)
        pltpu.sync_copy(tc_hbm, tab_c_v)
        for half in range(2):
            hbase = nbase + half * nss
            pltpu.sync_copy(an_hbm.at[pl.ds(hbase, nss)], an_v)
            pltpu.sync_copy(h_hbm.at[pl.ds(hbase, nss)], h_v)

            @plsc.parallel_loop(0, nss // _L, unroll=4)
            def _node(v):
                o = v * _L
                k16 = an_v[pl.ds(o, _L)] - 1
                h16 = h_v[pl.ds(o, _L)]
                a16 = plsc.load_gather(tab_a_v, [k16]) * h16
                c16 = plsc.load_gather(tab_c_v, [k16]) * h16 * h16
                sa_v[pl.ds(o, _L)] = a16
                sc_v[pl.ds(o, _L)] = c16

            pltpu.sync_copy(sa_v, a_hbm.at[pl.ds(cbase + hbase, nss)])
            pltpu.sync_copy(sc_v, c_hbm.at[pl.ds(cbase + hbase, nss)])

    pl.run_scoped(
        phase1,
        pltpu.VMEM((128,), f32),
        pltpu.VMEM((128,), f32),
        pltpu.VMEM((nss,), jnp.int32),
        pltpu.VMEM((nss,), f32),
        pltpu.VMEM((nss,), f32),
        pltpu.VMEM((nss,), f32),
    )

    @plsc.parallel_loop(0, nsl // _L, unroll=4)
    def _zero(v):
        pbuf[pl.ds(v * _L, _L)] = jnp.zeros((_L,), f32)

    pltpu.sync_copy(pbuf, accum.at[pl.ds(nbase, nsl)])
    plsc.subcore_barrier()

    pltpu.sync_copy(a_hbm.at[pl.ds(cbase, npad)], alpha_t)
    pltpu.sync_copy(c_hbm.at[pl.ds(cbase, npad)], c6_t)
    w = c * _NS + s
    ebase = w * ew

    def issue(b, ch):
        off = ebase + ch * _CHUNK
        pltpu.async_copy(ii_hbm.at[pl.ds(off, _CHUNK)], ii_b[b], sems.at[b])
        pltpu.async_copy(jj_hbm.at[pl.ds(off, _CHUNK)], jj_b[b], sems.at[b])
        pltpu.async_copy(dd_hbm.at[pl.ds(off, _CHUNK)], dd_b[b], sems.at[b])

    def drain(b, ch):
        off = ebase + ch * _CHUNK
        pltpu.make_async_copy(
            ii_hbm.at[pl.ds(off, _CHUNK)], ii_b[b], sems.at[b]).wait()
        pltpu.make_async_copy(
            jj_hbm.at[pl.ds(off, _CHUNK)], jj_b[b], sems.at[b]).wait()
        pltpu.make_async_copy(
            dd_hbm.at[pl.ds(off, _CHUNK)], dd_b[b], sems.at[b]).wait()

    def wait_scatter(b):
        pltpu.make_async_copy(ee_b[b], accum.at[ii_b[b]], ssems.at[b]).wait()

    def process_chunk(b, ch):
        nb = (b + 1) % 3
        nxt = ch + 1

        @pl.when(ch >= 2)
        def _retire():
            wait_scatter(nb)

        @pl.when(nxt < nch)
        def _issue_next():
            issue(nb, nxt)

        drain(b, ch)

        @plsc.parallel_loop(0, _CHUNK // _L, unroll=_UNROLL)
        def _edge(v):
            o = v * _L
            i16 = ii_b[b][pl.ds(o, _L)]
            j16 = jj_b[b][pl.ds(o, _L)]
            d16 = dd_b[b][pl.ds(o, _L)]
            ai = plsc.load_gather(alpha_t, [i16])
            aj = plsc.load_gather(alpha_t, [j16])
            ci = plsc.load_gather(c6_t, [i16])
            cj = plsc.load_gather(c6_t, [j16])
            ee_b[b][pl.ds(o, _L)] = _edge_energy(ai, aj, ci, cj, d16)

        pltpu.async_copy(ee_b[b], accum.at[ii_b[b]], ssems.at[b], add=True)

    issue(0, 0)

    def outer(g, _):
        for b in range(3):
            process_chunk(b, g * 3 + b)
        return _

    lax.fori_loop(0, nch // 3, outer, None)
    for ch in range((nch // 3) * 3, nch):
        process_chunk(ch % 3, ch)
    for ch in range(max(0, nch - 2), nch):
        wait_scatter(ch % 3)
    plsc.subcore_barrier()

    pltpu.sync_copy(accum.at[pl.ds(nbase, nsl)], pbuf)
    pltpu.sync_copy(pbuf, part_hbm.at[pl.ds(cbase + nbase, nsl)])


def _combine_body(p_ref, m_ref, o_ref):
    o_ref[...] = (p_ref[0] + p_ref[1]) * m_ref[...]


@jax.jit
def kernel(node_mask, atomic_numbers, idx_i_lr, idx_j_lr, d_ij_lr,
           hirshfeld_ratios):
    n = node_mask.shape[0]
    e = idx_i_lr.shape[0]
    nsl = ((n + _NS * _L - 1) // (_NS * _L)) * _L
    npad = _NS * nsl
    ew = ((e + _NW * _CHUNK - 1) // (_NW * _CHUNK)) * _CHUNK
    epad = _NW * ew
    nch = ew // _CHUNK

    an = jnp.pad(atomic_numbers.astype(jnp.int32), (0, npad - n),
                 constant_values=1)
    h = jnp.pad(hirshfeld_ratios.astype(jnp.float32), (0, npad - n))
    ii = jnp.pad(idx_i_lr.astype(jnp.int32), (0, epad - e))
    jj = jnp.pad(idx_j_lr.astype(jnp.int32), (0, epad - e))
    dd = jnp.pad(d_ij_lr.astype(jnp.float32), (0, epad - e))
    ta = jnp.asarray(np.pad(_ALPHAS_TAB, (0, 28)))
    tc = jnp.asarray(np.pad(_C6_TAB, (0, 28)))

    f32 = jnp.float32
    mesh = plsc.VectorSubcoreMesh(core_axis_name="c", subcore_axis_name="s")
    body = functools.partial(_sc_body, nsl, npad, ew, nch)
    parts, _, _ = pl.kernel(
        body,
        out_type=(
            jax.ShapeDtypeStruct((_NC * npad,), f32),
            jax.ShapeDtypeStruct((_NC * npad,), f32),
            jax.ShapeDtypeStruct((_NC * npad,), f32),
        ),
        mesh=mesh,
        compiler_params=pltpu.CompilerParams(needs_layout_passes=False),
        scratch_types=[
            pltpu.VMEM((npad,), f32),
            pltpu.VMEM((npad,), f32),
            pltpu.VMEM((_CHUNK,), jnp.int32),
            pltpu.VMEM((_CHUNK,), jnp.int32),
            pltpu.VMEM((_CHUNK,), jnp.int32),
            pltpu.VMEM((_CHUNK,), jnp.int32),
            pltpu.VMEM((_CHUNK,), jnp.int32),
            pltpu.VMEM((_CHUNK,), jnp.int32),
            pltpu.VMEM((_CHUNK,), f32),
            pltpu.VMEM((_CHUNK,), f32),
            pltpu.VMEM((_CHUNK,), f32),
            pltpu.VMEM((_CHUNK,), f32),
            pltpu.VMEM((_CHUNK,), f32),
            pltpu.VMEM((_CHUNK,), f32),
            pltpu.VMEM((nsl,), f32),
            pltpu.SemaphoreType.DMA((3,)),
            pltpu.SemaphoreType.DMA((3,)),
            pltpu.VMEM_SHARED((npad,), f32),
        ],
    )(an, h, ii, jj, dd, ta, tc)

    maskf = jnp.pad(node_mask.astype(f32), (0, npad - n))
    rows = npad // 128
    out = pl.pallas_call(
        _combine_body,
        out_shape=jax.ShapeDtypeStruct((rows, 128), f32),
    )(parts.reshape(_NC, rows, 128), maskf.reshape(rows, 128))
    return out.reshape(npad)[:n]


# speedup vs baseline: 1.2061x; 1.2061x over previous
import functools
import numpy as np
import jax
import jax.numpy as jnp
from jax import lax
from jax.experimental import pallas as pl
from jax.experimental.pallas import tpu as pltpu
from jax.experimental.pallas import tpu_sc as plsc

_FS = 0.0072973525693
_HARTREE = 27.211386245988
_BOHR = 0.529177210903
_XON = 8.0
_XOFF = 10.0

_C1 = float(_FS ** (-4.0 / 21.0))
_B0 = -0.00433008
_B1 = 0.24428889
_B2 = 0.04125273
_B3 = -0.00078893
_KI7 = float((8.0 / 7.0) * (127.0 - 0.0450466) * (2 ** 23))
_ZN = 1280
_ZBASE = 127 << 7
_ZK = np.arange(_ZN)
_ZLUT = ((2.0 ** (_ZK // 128)) * (1.0 + (_ZK % 128 + 0.5) / 128.0)) ** (-1.0 / 7.0)
_ZLUT = _ZLUT.astype(np.float32)

_NC, _NS, _L = 2, 16, 16
_NW = _NC * _NS
_CHUNK = 1024
_UNROLL = 4

_ALPHAS_TAB = np.linspace(4.5, 400.0, 100, dtype=np.float64).astype(np.float32)
_C6_TAB = np.linspace(6.5, 4000.0, 100, dtype=np.float64).astype(np.float32)


def _inv_root7(x):
    f32 = jnp.float32
    b = lax.bitcast_convert_type(x, jnp.int32).astype(f32)
    z = lax.bitcast_convert_type(
        (f32(_KI7) - b * f32(1.0 / 7.0)).astype(jnp.int32), f32)
    for _ in range(3):
        z2 = z * z
        z4 = z2 * z2
        xz7 = (x * z) * z2 * z4
        z = z * f32(8.0 / 7.0) - (z * xz7) * f32(1.0 / 7.0)
    return z


def _inv_root7_lut(x, zlut):
    f32 = jnp.float32
    bi = lax.bitcast_convert_type(x, jnp.int32)
    k = lax.shift_right_logical(bi, 16) - _ZBASE
    k = jnp.minimum(jnp.maximum(k, 0), _ZN - 1)
    z = plsc.load_gather(zlut, [k])
    z2 = z * z
    z4 = z2 * z2
    xz7 = (x * z) * z2 * z4
    return z * f32(8.0 / 7.0) - (z * xz7) * f32(1.0 / 7.0)


def _edge_energy(ai, aj, ci, cj, d, zlut):
    f32 = jnp.float32
    x = (ai + aj) * f32(0.5)
    c6 = (f32(2.0) * ci * cj * ai * aj) / (ai * ai * cj + aj * aj * ci)
    z = _inv_root7_lut(x, zlut)
    z2 = z * z
    z6 = z2 * z2 * z2
    t = x * z6
    vdw = f32(_C1) * t
    sig = ((f32(_B3) * vdw + f32(_B2)) * vdw + f32(_B1)) * vdw + f32(_B0)
    sig2 = sig * sig
    m8 = f32(10.0) * sig2
    m10 = f32(122.5) * sig2 * sig2
    p = f32(5.08) * t
    p2 = p * p
    p4 = p2 * p2
    r = d * f32(1.0 / _BOHR)
    r2 = r * r
    r4 = r2 * r2
    da = r4 * r2 + p4 * p2
    db = r4 * r4 + p4 * p4
    dc = r4 * r4 * r2 + p4 * p4 * p2
    dbdc = db * dc
    poly = dbdc + m8 * (da * dc) + m10 * (da * db)
    den3 = da * dbdc
    cc = (d - f32(_XON)) * f32(1.0 / (_XOFF - _XON))
    x1 = f32(1.0) - cc
    p1 = x1 > 0
    p2m = cc > 0
    x1p = jnp.where(p1, x1, f32(1.0))
    ccp = jnp.where(p2m, cc, f32(1.0))
    q = f32(1.0) / (x1p * ccp)
    s1 = jnp.where(p1, jnp.exp(-ccp * q), f32(0.0))
    s2 = jnp.where(p2m, jnp.exp(-x1p * q), f32(0.0))
    num = (c6 * s1) * poly
    den = den3 * (s1 + s2)
    e = num / den * f32(-0.5 * _HARTREE)
    return jnp.where(d > 0, e, f32(0.0))


def _sc_body(nsl, npad, ew, nch,
             an_hbm, h_hbm, ii_hbm, jj_hbm, dd_hbm, ta_hbm, tc_hbm, zl_hbm,
             part_hbm, a_hbm, c_hbm,
             alpha_t, c6_t, ii0_v, ii1_v, ii2_v, jj0_v, jj1_v, jj2_v,
             dd0_v, dd1_v, dd2_v, ee0_v, ee1_v, ee2_v, zlut_v, pbuf, sems,
             ssems, accum):
    ii_b = (ii0_v, ii1_v, ii2_v)
    jj_b = (jj0_v, jj1_v, jj2_v)
    dd_b = (dd0_v, dd1_v, dd2_v)
    ee_b = (ee0_v, ee1_v, ee2_v)
    c = lax.axis_index("c")
    s = lax.axis_index("s")
    f32 = jnp.float32
    nbase = s * nsl
    cbase = c * npad

    nss = nsl // 2

    def phase1(tab_a_v, tab_c_v, an_v, h_v, sa_v, sc_v):
        pltpu.sync_copy(ta_hbm, tab_a_v)
        pltpu.sync_copy(tc_hbm, tab_c_v)
        for half in range(2):
            hbase = nbase + half * nss
            pltpu.sync_copy(an_hbm.at[pl.ds(hbase, nss)], an_v)
            pltpu.sync_copy(h_hbm.at[pl.ds(hbase, nss)], h_v)

            @plsc.parallel_loop(0, nss // _L, unroll=4)
            def _node(v):
                o = v * _L
                k16 = an_v[pl.ds(o, _L)] - 1
                h16 = h_v[pl.ds(o, _L)]
                a16 = plsc.load_gather(tab_a_v, [k16]) * h16
                c16 = plsc.load_gather(tab_c_v, [k16]) * h16 * h16
                sa_v[pl.ds(o, _L)] = a16
                sc_v[pl.ds(o, _L)] = c16

            pltpu.sync_copy(sa_v, a_hbm.at[pl.ds(cbase + hbase, nss)])
            pltpu.sync_copy(sc_v, c_hbm.at[pl.ds(cbase + hbase, nss)])

    pl.run_scoped(
        phase1,
        pltpu.VMEM((128,), f32),
        pltpu.VMEM((128,), f32),
        pltpu.VMEM((nss,), jnp.int32),
        pltpu.VMEM((nss,), f32),
        pltpu.VMEM((nss,), f32),
        pltpu.VMEM((nss,), f32),
    )

    @plsc.parallel_loop(0, nsl // _L, unroll=4)
    def _zero(v):
        pbuf[pl.ds(v * _L, _L)] = jnp.zeros((_L,), f32)

    pltpu.sync_copy(pbuf, accum.at[pl.ds(nbase, nsl)])
    plsc.subcore_barrier()

    pltpu.sync_copy(zl_hbm, zlut_v)
    pltpu.sync_copy(a_hbm.at[pl.ds(cbase, npad)], alpha_t)
    pltpu.sync_copy(c_hbm.at[pl.ds(cbase, npad)], c6_t)
    w = c * _NS + s
    ebase = w * ew

    def issue(b, ch):
        off = ebase + ch * _CHUNK
        pltpu.async_copy(ii_hbm.at[pl.ds(off, _CHUNK)], ii_b[b], sems.at[b])
        pltpu.async_copy(jj_hbm.at[pl.ds(off, _CHUNK)], jj_b[b], sems.at[b])
        pltpu.async_copy(dd_hbm.at[pl.ds(off, _CHUNK)], dd_b[b], sems.at[b])

    def drain(b, ch):
        off = ebase + ch * _CHUNK
        pltpu.make_async_copy(
            ii_hbm.at[pl.ds(off, _CHUNK)], ii_b[b], sems.at[b]).wait()
        pltpu.make_async_copy(
            jj_hbm.at[pl.ds(off, _CHUNK)], jj_b[b], sems.at[b]).wait()
        pltpu.make_async_copy(
            dd_hbm.at[pl.ds(off, _CHUNK)], dd_b[b], sems.at[b]).wait()

    def wait_scatter(b):
        pltpu.make_async_copy(ee_b[b], accum.at[ii_b[b]], ssems.at[b]).wait()

    def process_chunk(b, ch):
        nb = (b + 1) % 3
        nxt = ch + 1

        @pl.when(ch >= 2)
        def _retire():
            wait_scatter(nb)

        @pl.when(nxt < nch)
        def _issue_next():
            issue(nb, nxt)

        drain(b, ch)

        @plsc.parallel_loop(0, _CHUNK // _L, unroll=_UNROLL)
        def _edge(v):
            o = v * _L
            i16 = ii_b[b][pl.ds(o, _L)]
            j16 = jj_b[b][pl.ds(o, _L)]
            d16 = dd_b[b][pl.ds(o, _L)]
            ai = plsc.load_gather(alpha_t, [i16])
            aj = plsc.load_gather(alpha_t, [j16])
            ci = plsc.load_gather(c6_t, [i16])
            cj = plsc.load_gather(c6_t, [j16])
            ee_b[b][pl.ds(o, _L)] = _edge_energy(ai, aj, ci, cj, d16,
                                                 zlut_v)

        pltpu.async_copy(ee_b[b], accum.at[ii_b[b]], ssems.at[b], add=True)

    issue(0, 0)

    def outer(g, _):
        for b in range(3):
            process_chunk(b, g * 3 + b)
        return _

    lax.fori_loop(0, nch // 3, outer, None)
    for ch in range((nch // 3) * 3, nch):
        process_chunk(ch % 3, ch)
    for ch in range(max(0, nch - 2), nch):
        wait_scatter(ch % 3)
    plsc.subcore_barrier()

    pltpu.sync_copy(accum.at[pl.ds(nbase, nsl)], pbuf)
    pltpu.sync_copy(pbuf, part_hbm.at[pl.ds(cbase + nbase, nsl)])


def _combine_body(p_ref, m_ref, o_ref):
    o_ref[...] = (p_ref[0] + p_ref[1]) * m_ref[...]


@jax.jit
def kernel(node_mask, atomic_numbers, idx_i_lr, idx_j_lr, d_ij_lr,
           hirshfeld_ratios):
    n = node_mask.shape[0]
    e = idx_i_lr.shape[0]
    nsl = ((n + _NS * _L - 1) // (_NS * _L)) * _L
    npad = _NS * nsl
    ew = ((e + _NW * _CHUNK - 1) // (_NW * _CHUNK)) * _CHUNK
    epad = _NW * ew
    nch = ew // _CHUNK

    an = jnp.pad(atomic_numbers.astype(jnp.int32), (0, npad - n),
                 constant_values=1)
    h = jnp.pad(hirshfeld_ratios.astype(jnp.float32), (0, npad - n))
    ii = jnp.pad(idx_i_lr.astype(jnp.int32), (0, epad - e))
    jj = jnp.pad(idx_j_lr.astype(jnp.int32), (0, epad - e))
    dd = jnp.pad(d_ij_lr.astype(jnp.float32), (0, epad - e))
    ta = jnp.asarray(np.pad(_ALPHAS_TAB, (0, 28)))
    tc = jnp.asarray(np.pad(_C6_TAB, (0, 28)))
    zl = jnp.asarray(_ZLUT)

    f32 = jnp.float32
    mesh = plsc.VectorSubcoreMesh(core_axis_name="c", subcore_axis_name="s")
    body = functools.partial(_sc_body, nsl, npad, ew, nch)
    parts, _, _ = pl.kernel(
        body,
        out_type=(
            jax.ShapeDtypeStruct((_NC * npad,), f32),
            jax.ShapeDtypeStruct((_NC * npad,), f32),
            jax.ShapeDtypeStruct((_NC * npad,), f32),
        ),
        mesh=mesh,
        compiler_params=pltpu.CompilerParams(needs_layout_passes=False),
        scratch_types=[
            pltpu.VMEM((npad,), f32),
            pltpu.VMEM((npad,), f32),
            pltpu.VMEM((_CHUNK,), jnp.int32),
            pltpu.VMEM((_CHUNK,), jnp.int32),
            pltpu.VMEM((_CHUNK,), jnp.int32),
            pltpu.VMEM((_CHUNK,), jnp.int32),
            pltpu.VMEM((_CHUNK,), jnp.int32),
            pltpu.VMEM((_CHUNK,), jnp.int32),
            pltpu.VMEM((_CHUNK,), f32),
            pltpu.VMEM((_CHUNK,), f32),
            pltpu.VMEM((_CHUNK,), f32),
            pltpu.VMEM((_CHUNK,), f32),
            pltpu.VMEM((_CHUNK,), f32),
            pltpu.VMEM((_CHUNK,), f32),
            pltpu.VMEM((_ZN,), f32),
            pltpu.VMEM((nsl,), f32),
            pltpu.SemaphoreType.DMA((3,)),
            pltpu.SemaphoreType.DMA((3,)),
            pltpu.VMEM_SHARED((npad,), f32),
        ],
    )(an, h, ii, jj, dd, ta, tc, zl)

    maskf = jnp.pad(node_mask.astype(f32), (0, npad - n))
    rows = npad // 128
    out = pl.pallas_call(
        _combine_body,
        out_shape=jax.ShapeDtypeStruct((rows, 128), f32),
    )(parts.reshape(_NC, rows, 128), maskf.reshape(rows, 128))
    return out.reshape(npad)[:n]
